# baseline (speedup 1.0000x reference)
import jax
import jax.numpy as jnp
from jax.experimental import pallas as pl
from jax.experimental.pallas import tpu as pltpu

_LQ = 200
_BB = 128


def _rosa_block(u_ref, x_ref, v_ref, o_ref):
    xq = x_ref[...].T
    vq = v_ref[...].T
    u = u_ref[0, 0]

    rows = jax.lax.broadcasted_iota(jnp.int32, (_LQ, 1), 0)
    out = jnp.full((_LQ, _BB), u, dtype=jnp.float32)
    for tp in range(_LQ - 1):
        xc = xq[tp:tp + 1, :]
        vc = vq[tp:tp + 1, :]
        m = (xq == xc) & (rows > tp)
        out = jnp.where(m, vc, out)
    o_ref[...] = out.T


def kernel(x, v, u):
    B, L = x.shape
    x32 = x.astype(jnp.int32)
    u_arr = jnp.full((1, 1), u, dtype=jnp.float32)

    out = pl.pallas_call(
        _rosa_block,
        grid=(B // _BB,),
        in_specs=[
            pl.BlockSpec(memory_space=pltpu.SMEM),
            pl.BlockSpec((_BB, L), lambda i: (i, 0)),
            pl.BlockSpec((_BB, L), lambda i: (i, 0)),
        ],
        out_specs=pl.BlockSpec((_BB, L), lambda i: (i, 0)),
        out_shape=jax.ShapeDtypeStruct((B, L), jnp.float32),
        compiler_params=pltpu.CompilerParams(
            dimension_semantics=("parallel",)),
    )(u_arr, x32, v)
    return out


# speedup vs baseline: 1.7670x; 1.7670x over previous
import jax
import jax.numpy as jnp
from jax.experimental import pallas as pl
from jax.experimental.pallas import tpu as pltpu

_LQ = 200
_BBL = 128


def _rosa_block(u_ref, x_ref, v_ref, o_ref):
    xq = x_ref[...]
    vq = v_ref[...]
    u = u_ref[0, 0]

    rows = jax.lax.broadcasted_iota(jnp.int32, (_LQ, 1), 0)
    out = jnp.full((_LQ, _BBL), u, dtype=jnp.float32)
    for tp in range(_LQ - 1):
        xc = xq[tp:tp + 1, :]
        vc = vq[tp:tp + 1, :]
        m = (xq == xc) & (rows > tp)
        out = jnp.where(m, vc, out)
    o_ref[...] = out


def kernel(x, v, u):
    B, L = x.shape
    xT = x.astype(jnp.int32).T
    vT = v.T
    u_arr = jnp.full((1, 1), u, dtype=jnp.float32)

    out = pl.pallas_call(
        _rosa_block,
        grid=(B // _BBL,),
        in_specs=[
            pl.BlockSpec(memory_space=pltpu.SMEM),
            pl.BlockSpec((L, _BBL), lambda i: (0, i)),
            pl.BlockSpec((L, _BBL), lambda i: (0, i)),
        ],
        out_specs=pl.BlockSpec((L, _BBL), lambda i: (0, i)),
        out_shape=jax.ShapeDtypeStruct((L, B), jnp.float32),
        compiler_params=pltpu.CompilerParams(
            dimension_semantics=("parallel",)),
    )(u_arr, xT, vT)
    return out.T


# speedup vs baseline: 1.8174x; 1.0285x over previous
import jax
import jax.numpy as jnp
from jax.experimental import pallas as pl
from jax.experimental.pallas import tpu as pltpu

_LQ = 200
_BBL = 256


def _rosa_block(u_ref, x_ref, v_ref, o_ref):
    xq = x_ref[...]
    vq = v_ref[...]
    u = u_ref[0, 0]

    rows = jax.lax.broadcasted_iota(jnp.int32, (_LQ, 1), 0)
    out = jnp.full((_LQ, _BBL), u, dtype=jnp.float32)
    for tp in range(_LQ - 1):
        xc = xq[tp:tp + 1, :]
        vc = vq[tp:tp + 1, :]
        m = (xq == xc) & (rows > tp)
        out = jnp.where(m, vc, out)
    o_ref[...] = out


def kernel(x, v, u):
    B, L = x.shape
    xT = x.astype(jnp.int32).T
    vT = v.T
    u_arr = jnp.full((1, 1), u, dtype=jnp.float32)

    out = pl.pallas_call(
        _rosa_block,
        grid=(B // _BBL,),
        in_specs=[
            pl.BlockSpec(memory_space=pltpu.SMEM),
            pl.BlockSpec((L, _BBL), lambda i: (0, i)),
            pl.BlockSpec((L, _BBL), lambda i: (0, i)),
        ],
        out_specs=pl.BlockSpec((L, _BBL), lambda i: (0, i)),
        out_shape=jax.ShapeDtypeStruct((L, B), jnp.float32),
        compiler_params=pltpu.CompilerParams(
            dimension_semantics=("parallel",)),
    )(u_arr, xT, vT)
    return out.T
